# baseline (speedup 1.0000x reference)
import functools

import jax
import jax.numpy as jnp
from jax import lax
from jax.experimental import pallas as pl
from jax.experimental.pallas import tpu as pltpu
from jax.experimental.pallas import tpu_sc as plsc

B = 16384
F = 64
NC = 2
NS = 16
NW = NC * NS
BPW = B // NW
CHUNK = 128
NCHUNK = BPW // CHUNK
LANES = 16
NGROUP = BPW // LANES


@functools.partial(
    pl.kernel,
    out_type=jax.ShapeDtypeStruct((B,), jnp.float32),
    mesh=plsc.VectorSubcoreMesh(core_axis_name="c", subcore_axis_name="s"),
    compiler_params=pltpu.CompilerParams(
        needs_layout_passes=False, use_tc_tiling_on_sc=False),
    scratch_types=[
        pltpu.VMEM((NCHUNK, CHUNK), jnp.int32),
        pltpu.VMEM((NCHUNK, CHUNK), jnp.int32),
        pltpu.VMEM((BPW, F), jnp.float32),
        pltpu.VMEM((BPW, F), jnp.float32),
        pltpu.VMEM((BPW,), jnp.float32),
        pltpu.VMEM((BPW,), jnp.float32),
        pltpu.VMEM((BPW,), jnp.float32),
        pltpu.SemaphoreType.DMA,
    ],
)
def _mf_kernel(user_ref, movie_ref, uf_ref, mf_ref, ub_ref, mb_ref,
               out_ref, idx_u, idx_m, uf_v, mf_v, ub_v, mb_v, out_v, sem):
    wid = lax.axis_index("s") * NC + lax.axis_index("c")

    pltpu.sync_copy(user_ref.at[wid], idx_u)
    pltpu.sync_copy(movie_ref.at[wid], idx_m)

    copies = []
    for k in range(NCHUNK):
        dst = pl.ds(k * CHUNK, CHUNK)
        copies.append(pltpu.async_copy(uf_ref.at[idx_u.at[k]], uf_v.at[dst], sem))
        copies.append(pltpu.async_copy(mf_ref.at[idx_m.at[k]], mf_v.at[dst], sem))
        copies.append(pltpu.async_copy(ub_ref.at[idx_u.at[k]], ub_v.at[dst], sem))
        copies.append(pltpu.async_copy(mb_ref.at[idx_m.at[k]], mb_v.at[dst], sem))
    for c in copies:
        c.wait()

    def group_body(g, carry):
        rows = g * LANES + lax.iota(jnp.int32, 16)
        acc0 = ub_v[pl.ds(g * LANES, LANES)] + mb_v[pl.ds(g * LANES, LANES)]

        def col_body(j, acc):
            cols = jnp.full((16,), j, jnp.int32)
            u = plsc.load_gather(uf_v, [rows, cols])
            m = plsc.load_gather(mf_v, [rows, cols])
            return acc + u * m

        acc = lax.fori_loop(0, F, col_body, acc0)
        out_v[pl.ds(g * LANES, LANES)] = acc
        return carry

    lax.fori_loop(0, NGROUP, group_body, 0)
    pltpu.sync_copy(out_v, out_ref.at[pl.ds(wid * BPW, BPW)])


@jax.jit
def kernel(user, movie, user_factors, movie_factors, user_biases, movie_biases):
    u3 = user.astype(jnp.int32).reshape(NW, NCHUNK, CHUNK)
    m3 = movie.astype(jnp.int32).reshape(NW, NCHUNK, CHUNK)
    ub = user_biases.reshape(-1)
    mb = movie_biases.reshape(-1)
    return _mf_kernel(u3, m3, user_factors, movie_factors, ub, mb)
